# baseline (speedup 1.0000x reference)
import functools

import jax
import jax.numpy as jnp
from jax import lax
from jax.experimental import pallas as pl
from jax.experimental.pallas import tpu as pltpu
from jax.experimental.pallas import tpu_sc as plsc

_CHUNK = 128
_L = 16


def _pack_body(tt_ref, out_ref):
    xt = jnp.transpose(tt_ref[...])
    xt3 = xt.reshape(128, 2, 64)
    out_ref[...] = jnp.concatenate([xt3[:, 0, :], xt3[:, 1, :]], axis=1)


def _pack_table(table_t):
    n_blocks = 50000 // 128
    grid = (391,)
    return pl.pallas_call(
        _pack_body,
        grid=grid,
        in_specs=[pl.BlockSpec((64, 256), lambda i: (0, i))],
        out_specs=pl.BlockSpec((128, 128), lambda i: (i, 0)),
        out_shape=jax.ShapeDtypeStruct((50048, 128), jnp.float32),
    )(table_t)


def _gather_body(table_hbm, idx_hbm, out_hbm, idx_v, pid_v, pair_v, sem,
                 *, nc, b_per_w):
    wid = lax.axis_index("s") * nc + lax.axis_index("c")
    base = wid * b_per_w
    n_chunks = b_per_w // _CHUNK
    pltpu.sync_copy(idx_hbm.at[pl.ds(base, b_per_w)], idx_v)
    for k in range(b_per_w // _L):
        v = idx_v[pl.ds(k * _L, _L)]
        pid_v[pl.ds(k * _L, _L)] = lax.shift_right_logical(v, 1)
    copies = []
    for j in range(n_chunks):
        copies.append(
            pltpu.async_copy(
                table_hbm.at[pid_v.at[pl.ds(j * _CHUNK, _CHUNK)]],
                pair_v.at[pl.ds(j * _CHUNK, _CHUNK)],
                sem,
            )
        )
    for c in copies:
        c.wait()
    pltpu.sync_copy(pair_v, out_hbm.at[pl.ds(base, b_per_w)])


def kernel(positional_encoding, time_steps):
    V, D = positional_encoding.shape
    (B,) = time_steps.shape
    info = plsc.get_sparse_core_info()
    nc, ns = info.num_cores, info.num_subcores
    nw = nc * ns
    b_per_w = B // nw
    table_t = jnp.transpose(positional_encoding)
    table2 = _pack_table(table_t)
    mesh = plsc.VectorSubcoreMesh(core_axis_name="c", subcore_axis_name="s")
    run = pl.kernel(
        functools.partial(_gather_body, nc=nc, b_per_w=b_per_w),
        mesh=mesh,
        out_type=jax.ShapeDtypeStruct((B, 2 * D), positional_encoding.dtype),
        scratch_types=[
            pltpu.VMEM((b_per_w,), jnp.int32),
            pltpu.VMEM((b_per_w,), jnp.int32),
            pltpu.VMEM((b_per_w, 2 * D), jnp.float32),
            pltpu.SemaphoreType.DMA,
        ],
        compiler_params=pltpu.CompilerParams(use_tc_tiling_on_sc=True),
    )
    pairs = run(table2, time_steps)
    half = (time_steps & 1)[:, None]
    return jnp.where(half == 1, pairs[:, D:], pairs[:, :D])
